# baseline (speedup 1.0000x reference)
import functools

import numpy as np
import jax
import jax.numpy as jnp
from jax import lax
from jax.experimental import pallas as pl
from jax.experimental.pallas import tpu as pltpu
from jax.experimental.pallas import tpu_sc as plsc

VOCAB = 100000
D = 128
SEQ = 200
B = 1024

N = B * SEQ
NC = 2
NS = 16
NW = NC * NS
RPW = N // NW
CHUNK = 80
NCHUNK = RPW // CHUNK
NBUF = 4
LEAD = 2
GROUP = 20
NGROUP = NCHUNK // GROUP
LANES = 16
PE_EXT = SEQ + CHUNK


def _make_pe():
    pos = np.arange(SEQ, dtype=np.float64)[:, None]
    div = np.exp(np.arange(0, D, 2, dtype=np.float64) * -(np.log(10000.0) / D))
    pe = np.zeros((SEQ, D), dtype=np.float32)
    pe[:, 0::2] = np.sin(pos * div).astype(np.float32)
    pe[:, 1::2] = np.cos(pos * div).astype(np.float32)
    return pe


def _make_pe_ext():
    pe = _make_pe()
    return jnp.asarray(np.concatenate([pe, pe[:CHUNK]], axis=0))


_MESH = plsc.VectorSubcoreMesh(
    core_axis_name="c", subcore_axis_name="s", num_cores=NC, num_subcores=NS
)


@functools.partial(
    pl.kernel,
    mesh=_MESH,
    out_type=jax.ShapeDtypeStruct((N, D), jnp.float32),
    scratch_types=[
        pltpu.VMEM((NCHUNK, CHUNK), jnp.int32),
        pltpu.VMEM((PE_EXT, D), jnp.float32),
        pltpu.VMEM((NBUF, CHUNK, D), jnp.float32),
    ]
    + [pltpu.SemaphoreType.DMA] * (2 * NBUF),
)
def _embed_sc(table_hbm, idx_hbm, pe_hbm, out_hbm, idx_v, pe_v, bufs, *sems):
    gsems = sems[:NBUF]
    ssems = sems[NBUF:]

    wid = lax.axis_index("s") * NC + lax.axis_index("c")
    base = wid * RPW

    pltpu.sync_copy(idx_hbm.at[wid], idx_v)

    def gather(c, b):
        return pltpu.make_async_copy(
            table_hbm.at[idx_v.at[c]], bufs.at[b], gsems[b]
        )

    def store(c, b):
        return pltpu.make_async_copy(
            bufs.at[b], out_hbm.at[pl.ds(base + c * CHUNK, CHUNK)], ssems[b]
        )

    HALF = CHUNK // 2

    def store_half(c, b, h):
        return pltpu.make_async_copy(
            bufs.at[b, pl.ds(h * HALF, HALF)],
            out_hbm.at[pl.ds(base + c * CHUNK + h * HALF, HALF)],
            ssems[b],
        )

    for c0 in range(LEAD):
        gather(c0, c0).start()
    pltpu.sync_copy(pe_hbm, pe_v)

    def group(g, carry):
        for k in range(GROUP):
            c = g * GROUP + k
            b = k % NBUF
            gather(c, b).wait()

            nxt = c + LEAD
            bn = (k + LEAD) % NBUF
            if k < NBUF - LEAD:
                @pl.when(g >= 1)
                def _():
                    store(nxt - NBUF, bn).wait()

                gather(nxt, bn).start()
            elif k < GROUP - LEAD:
                store(nxt - NBUF, bn).wait()
                gather(nxt, bn).start()
            else:
                @pl.when(g < NGROUP - 1)
                def _():
                    store(nxt - NBUF, bn).wait()
                    gather(nxt, bn).start()

            peo = (k * CHUNK) % SEQ

            for h in range(2):
                @plsc.parallel_loop(h * HALF, (h + 1) * HALF, step=1, unroll=4)
                def add_row(r):
                    for j in range(D // LANES):
                        v = pe_v[peo + r, pl.ds(j * LANES, LANES)]
                        plsc.addupdate(bufs.at[b, r, pl.ds(j * LANES, LANES)], v)

                store_half(c, b, h).start()
        return carry

    lax.fori_loop(0, NGROUP, group, 0)

    for c0 in range(NCHUNK - NBUF, NCHUNK):
        store(c0, c0 % NBUF).wait()


def kernel(sequence, token_table):
    idx = sequence.astype(jnp.int32).reshape(NW, NCHUNK, CHUNK)
    pe = _make_pe_ext()
    out = _embed_sc(token_table, idx, pe)
    return out.reshape(B, SEQ, D)


# speedup vs baseline: 1.0471x; 1.0471x over previous
import functools

import numpy as np
import jax
import jax.numpy as jnp
from jax import lax
from jax.experimental import pallas as pl
from jax.experimental.pallas import tpu as pltpu
from jax.experimental.pallas import tpu_sc as plsc

VOCAB = 100000
D = 128
SEQ = 200
B = 1024

N = B * SEQ
NC = 2
NS = 16
NW = NC * NS
RPW = N // NW
CHUNK = 80
NCHUNK = RPW // CHUNK
NBUF = 8
LEAD = 4
GROUP = 40
NGROUP = NCHUNK // GROUP
LANES = 16
PE_EXT = SEQ + CHUNK


def _make_pe():
    pos = np.arange(SEQ, dtype=np.float64)[:, None]
    div = np.exp(np.arange(0, D, 2, dtype=np.float64) * -(np.log(10000.0) / D))
    pe = np.zeros((SEQ, D), dtype=np.float32)
    pe[:, 0::2] = np.sin(pos * div).astype(np.float32)
    pe[:, 1::2] = np.cos(pos * div).astype(np.float32)
    return pe


def _make_pe_ext():
    pe = _make_pe()
    return jnp.asarray(np.concatenate([pe, pe[:CHUNK]], axis=0))


_MESH = plsc.VectorSubcoreMesh(
    core_axis_name="c", subcore_axis_name="s", num_cores=NC, num_subcores=NS
)


@functools.partial(
    pl.kernel,
    mesh=_MESH,
    out_type=jax.ShapeDtypeStruct((N, D), jnp.float32),
    scratch_types=[
        pltpu.VMEM((NCHUNK, CHUNK), jnp.int32),
        pltpu.VMEM((PE_EXT, D), jnp.float32),
        pltpu.VMEM((NBUF, CHUNK, D), jnp.float32),
    ]
    + [pltpu.SemaphoreType.DMA] * (2 * NBUF),
)
def _embed_sc(table_hbm, idx_hbm, pe_hbm, out_hbm, idx_v, pe_v, bufs, *sems):
    gsems = sems[:NBUF]
    ssems = sems[NBUF:]

    wid = lax.axis_index("s") * NC + lax.axis_index("c")
    base = wid * RPW

    pltpu.sync_copy(idx_hbm.at[wid], idx_v)

    def gather(c, b):
        return pltpu.make_async_copy(
            table_hbm.at[idx_v.at[c]], bufs.at[b], gsems[b]
        )

    def store(c, b):
        return pltpu.make_async_copy(
            bufs.at[b], out_hbm.at[pl.ds(base + c * CHUNK, CHUNK)], ssems[b]
        )

    HALF = CHUNK // 2

    def store_half(c, b, h):
        return pltpu.make_async_copy(
            bufs.at[b, pl.ds(h * HALF, HALF)],
            out_hbm.at[pl.ds(base + c * CHUNK + h * HALF, HALF)],
            ssems[b],
        )

    for c0 in range(LEAD):
        gather(c0, c0).start()
    pltpu.sync_copy(pe_hbm, pe_v)

    def group(g, carry):
        for k in range(GROUP):
            c = g * GROUP + k
            b = k % NBUF
            gather(c, b).wait()

            nxt = c + LEAD
            bn = (k + LEAD) % NBUF
            if k < NBUF - LEAD:
                @pl.when(g >= 1)
                def _():
                    store(nxt - NBUF, bn).wait()

                gather(nxt, bn).start()
            elif k < GROUP - LEAD:
                store(nxt - NBUF, bn).wait()
                gather(nxt, bn).start()
            else:
                @pl.when(g < NGROUP - 1)
                def _():
                    store(nxt - NBUF, bn).wait()
                    gather(nxt, bn).start()

            peo = (k * CHUNK) % SEQ

            for h in range(2):
                @plsc.parallel_loop(h * HALF, (h + 1) * HALF, step=1, unroll=2)
                def add_row(r):
                    for j in range(D // LANES):
                        v = pe_v[peo + r, pl.ds(j * LANES, LANES)]
                        plsc.addupdate(bufs.at[b, r, pl.ds(j * LANES, LANES)], v)

                store_half(c, b, h).start()
        return carry

    lax.fori_loop(0, NGROUP, group, 0)

    for c0 in range(NCHUNK - NBUF, NCHUNK):
        store(c0, c0 % NBUF).wait()


def kernel(sequence, token_table):
    idx = sequence.astype(jnp.int32).reshape(NW, NCHUNK, CHUNK)
    pe = _make_pe_ext()
    out = _embed_sc(token_table, idx, pe)
    return out.reshape(B, SEQ, D)
